# Initial kernel scaffold; baseline (speedup 1.0000x reference)
import jax
import jax.numpy as jnp
from jax.experimental import pallas as pl


def kernel(x, edge_index, edge_attr, W_in, b_in, We, be, W1, b1, W2, b2, gamma, beta, Wr1, br1, Wr2, br2):
    raise NotImplementedError("write your pallas kernel here")


# speedup vs baseline: 2.1331x; 2.1331x over previous
import functools

import jax
import jax.numpy as jnp
from jax import lax
from jax.experimental import pallas as pl
from jax.experimental.pallas import tpu as pltpu
from jax.experimental.pallas import tpu_sc as plsc

_N = 50000
_E = 800000
_D_IN = 176
_H = 64
_HQ = 16
_L = 4

_EPAD = 819200
_ROWS = _EPAD // 128
_NSUB = 16
_RPT = _ROWS // _NSUB
_CROWS = 16
_CHUNK = _CROWS * 128
_NCHUNK = _RPT // _CROWS
_AGGR = 50176
_STRIPE = _AGGR // _NSUB
_ZROWS = 784
_ZREP = _STRIPE // _ZROWS

_BR = 2000


def _sc_layer_body(q, h_hbm, src_hbm, dst_hbm, ea_hbm, wb_hbm, out_hbm,
                   gbuf, eabuf, srcv, dstv, wbuf, aggsp, gsem, ssem):
    c = lax.axis_index("c")
    s = lax.axis_index("s")

    pltpu.sync_copy(wb_hbm.at[pl.ds(c * 8, 8)], wbuf)
    w = [wbuf[k] for k in range(5)]

    zv = jnp.zeros((16,), jnp.float32)

    def zrow(i, carry):
        gbuf[i] = zv
        return carry

    lax.fori_loop(0, _ZROWS, zrow, 0)
    base = s * _STRIPE
    for t in range(_ZREP):
        pltpu.sync_copy(gbuf.at[pl.ds(0, _ZROWS)],
                        aggsp.at[pl.ds(base + t * _ZROWS, _ZROWS)])
    plsc.subcore_barrier()

    row0 = s * _RPT
    goff = ((2 * q + c) * _N).astype(jnp.int32)

    def chunk(g, carry):
        rb = row0 + g * _CROWS
        pltpu.sync_copy(src_hbm.at[pl.ds(rb, _CROWS)], srcv)
        pltpu.sync_copy(dst_hbm.at[pl.ds(rb, _CROWS)], dstv)
        pltpu.sync_copy(ea_hbm.at[pl.ds(rb * 32, _CHUNK // 4)], eabuf)

        def adjrow(j, cr):
            for jj in range(8):
                sl = pl.ds(jj * 16, 16)
                srcv[j, sl] = srcv[j, sl] + goff
            return cr

        lax.fori_loop(0, _CROWS, adjrow, 0)

        cps = [pltpu.async_copy(h_hbm.at[srcv.at[j]],
                                gbuf.at[pl.ds(j * 128, 128)], gsem)
               for j in range(_CROWS)]
        for cp in cps:
            cp.wait()

        def edge4(g4, cr):
            eav = eabuf[g4]
            for i in range(4):
                e = g4 * 4 + i
                acc = w[4]
                for k in range(4):
                    acc = acc + eav[4 * i + k] * w[k]
                gbuf[e] = jnp.maximum(acc + gbuf[e], 0.0)
            return cr

        lax.fori_loop(0, _CHUNK // 4, edge4, 0)

        cps2 = [pltpu.async_copy(gbuf.at[pl.ds(j * 128, 128)],
                                 aggsp.at[dstv.at[j]], ssem, add=True)
                for j in range(_CROWS)]
        for cp in cps2:
            cp.wait()
        return carry

    lax.fori_loop(0, _NCHUNK, chunk, 0)
    plsc.subcore_barrier()

    obase = c * _AGGR + s * _STRIPE
    for t in range(_ZREP):
        pltpu.sync_copy(aggsp.at[pl.ds(base + t * _ZROWS, _ZROWS)],
                        gbuf.at[pl.ds(0, _ZROWS)])
        pltpu.sync_copy(gbuf.at[pl.ds(0, _ZROWS)],
                        out_hbm.at[pl.ds(obase + t * _ZROWS, _ZROWS)])


def _make_sc_layer(q):
    return pl.kernel(
        functools.partial(_sc_layer_body, q),
        out_type=jax.ShapeDtypeStruct((2 * _AGGR, _HQ), jnp.float32),
        mesh=plsc.VectorSubcoreMesh(core_axis_name="c", subcore_axis_name="s",
                                    num_cores=2, num_subcores=_NSUB),
        scratch_types=[
            pltpu.VMEM((_CHUNK, _HQ), jnp.float32),
            pltpu.VMEM((_CHUNK // 4, 16), jnp.float32),
            pltpu.VMEM((_CROWS, 128), jnp.int32),
            pltpu.VMEM((_CROWS, 128), jnp.int32),
            pltpu.VMEM((8, _HQ), jnp.float32),
            pltpu.VMEM_SHARED((_AGGR, _HQ), jnp.float32),
            pltpu.SemaphoreType.DMA,
            pltpu.SemaphoreType.DMA,
        ],
        compiler_params=pltpu.CompilerParams(use_tc_tiling_on_sc=False),
    )


_sc_layer_q0 = _make_sc_layer(0)
_sc_layer_q1 = _make_sc_layer(1)


def _tc_in_body(x_ref, w_ref, b_ref, out_ref):
    h = jnp.dot(x_ref[...], w_ref[...],
                preferred_element_type=jnp.float32) + b_ref[...]
    for t in range(4):
        out_ref[t] = h[:, t * _HQ:(t + 1) * _HQ]


_tc_in = pl.pallas_call(
    _tc_in_body,
    grid=(_N // _BR,),
    in_specs=[
        pl.BlockSpec((_BR, _D_IN), lambda i: (i, 0)),
        pl.BlockSpec((_D_IN, _H), lambda i: (0, 0)),
        pl.BlockSpec((1, _H), lambda i: (0, 0)),
    ],
    out_specs=pl.BlockSpec((4, _BR, _HQ), lambda i: (0, i, 0)),
    out_shape=jax.ShapeDtypeStruct((4, _N, _HQ), jnp.float32),
)


def _node_update(h_ref, agga_ref, aggb_ref, w1_ref, b1_ref, w2_ref, b2_ref,
                 gm_ref, bt_ref):
    z = jnp.concatenate(
        [h_ref[0] + agga_ref[0], h_ref[1] + agga_ref[1],
         h_ref[2] + aggb_ref[0], h_ref[3] + aggb_ref[1]], axis=1)
    t = jnp.maximum(jnp.dot(z, w1_ref[...],
                            preferred_element_type=jnp.float32)
                    + b1_ref[...], 0.0)
    z2 = jnp.dot(t, w2_ref[...],
                 preferred_element_type=jnp.float32) + b2_ref[...]
    mu = jnp.mean(z2, axis=1, keepdims=True)
    var = jnp.mean((z2 - mu) ** 2, axis=1, keepdims=True)
    zn = (z2 - mu) * lax.rsqrt(var + 1e-5)
    return jnp.maximum(zn * gm_ref[...] + bt_ref[...], 0.0)


def _tc_up_body(h_ref, agga_ref, aggb_ref, w1_ref, b1_ref, w2_ref, b2_ref,
                gm_ref, bt_ref, out_ref):
    hn = _node_update(h_ref, agga_ref, aggb_ref, w1_ref, b1_ref, w2_ref,
                      b2_ref, gm_ref, bt_ref)
    for t in range(4):
        out_ref[t] = hn[:, t * _HQ:(t + 1) * _HQ]


def _tc_up_last_body(h_ref, agga_ref, aggb_ref, w1_ref, b1_ref, w2_ref,
                     b2_ref, gm_ref, bt_ref, out_ref):
    hn = _node_update(h_ref, agga_ref, aggb_ref, w1_ref, b1_ref, w2_ref,
                      b2_ref, gm_ref, bt_ref)

    @pl.when(pl.program_id(0) == 0)
    def _():
        out_ref[...] = jnp.zeros_like(out_ref)

    out_ref[...] += jnp.sum(hn, axis=0, keepdims=True)


_up_in_specs = [
    pl.BlockSpec((4, _BR, _HQ), lambda i: (0, i, 0)),
    pl.BlockSpec((2, _BR, _HQ), lambda i: (0, i, 0)),
    pl.BlockSpec((2, _BR, _HQ), lambda i: (0, i, 0)),
    pl.BlockSpec((_H, 2 * _H), lambda i: (0, 0)),
    pl.BlockSpec((1, 2 * _H), lambda i: (0, 0)),
    pl.BlockSpec((2 * _H, _H), lambda i: (0, 0)),
    pl.BlockSpec((1, _H), lambda i: (0, 0)),
    pl.BlockSpec((1, _H), lambda i: (0, 0)),
    pl.BlockSpec((1, _H), lambda i: (0, 0)),
]

_tc_up = pl.pallas_call(
    _tc_up_body,
    grid=(_N // _BR,),
    in_specs=_up_in_specs,
    out_specs=pl.BlockSpec((4, _BR, _HQ), lambda i: (0, i, 0)),
    out_shape=jax.ShapeDtypeStruct((4, _N, _HQ), jnp.float32),
)

_tc_up_last = pl.pallas_call(
    _tc_up_last_body,
    grid=(_N // _BR,),
    in_specs=_up_in_specs,
    out_specs=pl.BlockSpec((1, _H), lambda i: (0, 0)),
    out_shape=jax.ShapeDtypeStruct((1, _H), jnp.float32),
)


def _tc_head_body(g_ref, wr1_ref, br1_ref, wr2_ref, br2_ref, out_ref):
    t = jnp.maximum(jnp.dot(g_ref[...], wr1_ref[...],
                            preferred_element_type=jnp.float32)
                    + br1_ref[...], 0.0)
    out_ref[...] = jnp.dot(t, wr2_ref[...],
                           preferred_element_type=jnp.float32) + br2_ref[...]


_tc_head = pl.pallas_call(
    _tc_head_body,
    out_shape=jax.ShapeDtypeStruct((1, 1), jnp.float32),
)


def _edge_weights(We_l, be_l, q):
    zpad = jnp.zeros((3, _HQ), jnp.float32)
    lo = q * 2 * _HQ
    return jnp.concatenate(
        [We_l[:, lo:lo + _HQ], be_l[None, lo:lo + _HQ], zpad,
         We_l[:, lo + _HQ:lo + 2 * _HQ], be_l[None, lo + _HQ:lo + 2 * _HQ],
         zpad], axis=0)


def kernel(x, edge_index, edge_attr, W_in, b_in, We, be, W1, b1, W2, b2,
           gamma, beta, Wr1, br1, Wr2, br2):
    pad = _EPAD - _E
    src2d = jnp.concatenate(
        [edge_index[0], jnp.zeros((pad,), jnp.int32)]).reshape(_ROWS, 128)
    dst2d = jnp.concatenate(
        [edge_index[1], jnp.full((pad,), _N, jnp.int32)]).reshape(_ROWS, 128)
    ea_pad = jnp.concatenate(
        [edge_attr, jnp.zeros((pad, 4), jnp.float32)],
        axis=0).reshape(_EPAD // 4, 16)

    h4 = _tc_in(x, W_in, b_in.reshape(1, _H))
    g = None
    for l in range(_L):
        hflat = h4.reshape(4 * _N, _HQ)
        agg_a = _sc_layer_q0(hflat, src2d, dst2d, ea_pad,
                             _edge_weights(We[l], be[l], 0))
        agg_b = _sc_layer_q1(hflat, src2d, dst2d, ea_pad,
                             _edge_weights(We[l], be[l], 1))
        args = (h4, agg_a.reshape(2, _AGGR, _HQ), agg_b.reshape(2, _AGGR, _HQ),
                W1[l], b1[l].reshape(1, -1), W2[l], b2[l].reshape(1, -1),
                gamma[l].reshape(1, -1), beta[l].reshape(1, -1))
        if l < _L - 1:
            h4 = _tc_up(*args)
        else:
            g = _tc_up_last(*args)
    out = _tc_head(g, Wr1, br1.reshape(1, -1), Wr2, br2.reshape(1, -1))
    return out.reshape(())


# speedup vs baseline: 3.0095x; 1.4109x over previous
import functools

import jax
import jax.numpy as jnp
from jax import lax
from jax.experimental import pallas as pl
from jax.experimental.pallas import tpu as pltpu
from jax.experimental.pallas import tpu_sc as plsc

_N = 50000
_E = 800000
_D_IN = 176
_H = 64
_HQ = 16
_L = 4

_EPAD = 819200
_ROWS = _EPAD // 128
_NSUB = 16
_RPT = _ROWS // _NSUB
_CROWS = 8
_CHUNK = _CROWS * 128
_NCHUNK = _RPT // _CROWS
_AGGR = 50176
_STRIPE = _AGGR // _NSUB
_ZROWS = 784
_ZREP = _STRIPE // _ZROWS

_BR = 2000


def _sc_layer_body(q, h_hbm, src_hbm, dst_hbm, ea_hbm, wb_hbm, out_hbm,
                   gbuf0, gbuf1, ea0, ea1, sv0, sv1, dv0, dv1, wbuf, aggsp,
                   gsem0, gsem1, ssem0, ssem1):
    c = lax.axis_index("c")
    s = lax.axis_index("s")

    pltpu.sync_copy(wb_hbm.at[pl.ds(c * 8, 8)], wbuf)
    w = [wbuf[k] for k in range(5)]

    zv = jnp.zeros((16,), jnp.float32)

    def zrow(i, carry):
        gbuf0[i] = zv
        return carry

    lax.fori_loop(0, _ZROWS, zrow, 0)
    base = s * _STRIPE
    for t in range(_ZREP):
        pltpu.sync_copy(gbuf0.at[pl.ds(0, _ZROWS)],
                        aggsp.at[pl.ds(base + t * _ZROWS, _ZROWS)])
    plsc.subcore_barrier()

    row0 = s * _RPT
    goff = ((2 * q + c) * _N).astype(jnp.int32)

    set0 = (gbuf0, ea0, sv0, dv0, gsem0, ssem0)
    set1 = (gbuf1, ea1, sv1, dv1, gsem1, ssem1)

    def prefetch(g, bufs):
        gbuf, eab, sv, dv, gsem, _ = bufs
        rb = row0 + g * _CROWS
        pltpu.sync_copy(src_hbm.at[pl.ds(rb, _CROWS)], sv)
        pltpu.sync_copy(dst_hbm.at[pl.ds(rb, _CROWS)], dv)
        for k in range(4):
            pltpu.sync_copy(ea_hbm.at[pl.ds(k * _EPAD + rb * 128, _CHUNK)],
                            eab.at[k])

        def adjrow(j, cr):
            for jj in range(8):
                sl = pl.ds(jj * 16, 16)
                sv[j, sl] = sv[j, sl] + goff
            return cr

        lax.fori_loop(0, _CROWS, adjrow, 0)
        for j in range(_CROWS):
            pltpu.async_copy(h_hbm.at[sv.at[j]],
                             gbuf.at[pl.ds(j * 128, 128)], gsem)

    def wait_n(sem, gbuf):
        for j in range(_CROWS):
            pltpu.make_async_copy(h_hbm.at[pl.ds(0, 128)],
                                  gbuf.at[pl.ds(0, 128)], sem).wait()

    def process(bufs):
        gbuf, eab, sv, dv, gsem, ssem = bufs
        wait_n(gsem, gbuf)

        def edge16(t, cr):
            e0 = t * 16
            v = [eab[k, pl.ds(e0, 16)] for k in range(4)]
            for i in range(16):
                acc = w[4]
                for k in range(4):
                    acc = acc + v[k][i] * w[k]
                e = e0 + i
                gbuf[e] = jnp.maximum(acc + gbuf[e], 0.0)
            return cr

        lax.fori_loop(0, _CHUNK // 16, edge16, 0)
        for j in range(_CROWS):
            pltpu.async_copy(gbuf.at[pl.ds(j * 128, 128)],
                             aggsp.at[dv.at[j]], ssem, add=True)

    def drain(bufs):
        gbuf, _, _, _, _, ssem = bufs
        wait_n(ssem, gbuf)

    prefetch(0, set0)
    prefetch(1, set1)
    process(set0)

    def pairbody(i, cr):
        t = 1 + 2 * i
        drain(set0)
        prefetch(t + 1, set0)
        process(set1)
        drain(set1)
        prefetch(t + 2, set1)
        process(set0)
        return cr

    lax.fori_loop(0, (_NCHUNK - 2) // 2, pairbody, 0)
    process(set1)
    drain(set0)
    drain(set1)
    plsc.subcore_barrier()

    obase = c * _AGGR + s * _STRIPE
    for t in range(_ZREP):
        pltpu.sync_copy(aggsp.at[pl.ds(base + t * _ZROWS, _ZROWS)],
                        gbuf0.at[pl.ds(0, _ZROWS)])
        pltpu.sync_copy(gbuf0.at[pl.ds(0, _ZROWS)],
                        out_hbm.at[pl.ds(obase + t * _ZROWS, _ZROWS)])


def _make_sc_layer(q):
    return pl.kernel(
        functools.partial(_sc_layer_body, q),
        out_type=jax.ShapeDtypeStruct((2 * _AGGR, _HQ), jnp.float32),
        mesh=plsc.VectorSubcoreMesh(core_axis_name="c", subcore_axis_name="s",
                                    num_cores=2, num_subcores=_NSUB),
        scratch_types=[
            pltpu.VMEM((_CHUNK, _HQ), jnp.float32),
            pltpu.VMEM((_CHUNK, _HQ), jnp.float32),
            pltpu.VMEM((4, _CHUNK), jnp.float32),
            pltpu.VMEM((4, _CHUNK), jnp.float32),
            pltpu.VMEM((_CROWS, 128), jnp.int32),
            pltpu.VMEM((_CROWS, 128), jnp.int32),
            pltpu.VMEM((_CROWS, 128), jnp.int32),
            pltpu.VMEM((_CROWS, 128), jnp.int32),
            pltpu.VMEM((8, _HQ), jnp.float32),
            pltpu.VMEM_SHARED((_AGGR, _HQ), jnp.float32),
            pltpu.SemaphoreType.DMA,
            pltpu.SemaphoreType.DMA,
            pltpu.SemaphoreType.DMA,
            pltpu.SemaphoreType.DMA,
        ],
        compiler_params=pltpu.CompilerParams(use_tc_tiling_on_sc=False),
    )


_sc_layer_q0 = _make_sc_layer(0)
_sc_layer_q1 = _make_sc_layer(1)


def _tc_in_body(x_ref, w_ref, b_ref, out_ref):
    h = jnp.dot(x_ref[...], w_ref[...],
                preferred_element_type=jnp.float32) + b_ref[...]
    for t in range(4):
        out_ref[t] = h[:, t * _HQ:(t + 1) * _HQ]


_tc_in = pl.pallas_call(
    _tc_in_body,
    grid=(_N // _BR,),
    in_specs=[
        pl.BlockSpec((_BR, _D_IN), lambda i: (i, 0)),
        pl.BlockSpec((_D_IN, _H), lambda i: (0, 0)),
        pl.BlockSpec((1, _H), lambda i: (0, 0)),
    ],
    out_specs=pl.BlockSpec((4, _BR, _HQ), lambda i: (0, i, 0)),
    out_shape=jax.ShapeDtypeStruct((4, _N, _HQ), jnp.float32),
)


def _node_update(h_ref, agga_ref, aggb_ref, w1_ref, b1_ref, w2_ref, b2_ref,
                 gm_ref, bt_ref):
    z = jnp.concatenate(
        [h_ref[0] + agga_ref[0], h_ref[1] + agga_ref[1],
         h_ref[2] + aggb_ref[0], h_ref[3] + aggb_ref[1]], axis=1)
    t = jnp.maximum(jnp.dot(z, w1_ref[...],
                            preferred_element_type=jnp.float32)
                    + b1_ref[...], 0.0)
    z2 = jnp.dot(t, w2_ref[...],
                 preferred_element_type=jnp.float32) + b2_ref[...]
    mu = jnp.mean(z2, axis=1, keepdims=True)
    var = jnp.mean((z2 - mu) ** 2, axis=1, keepdims=True)
    zn = (z2 - mu) * lax.rsqrt(var + 1e-5)
    return jnp.maximum(zn * gm_ref[...] + bt_ref[...], 0.0)


def _tc_up_body(h_ref, agga_ref, aggb_ref, w1_ref, b1_ref, w2_ref, b2_ref,
                gm_ref, bt_ref, out_ref):
    hn = _node_update(h_ref, agga_ref, aggb_ref, w1_ref, b1_ref, w2_ref,
                      b2_ref, gm_ref, bt_ref)
    for t in range(4):
        out_ref[t] = hn[:, t * _HQ:(t + 1) * _HQ]


def _tc_up_last_body(h_ref, agga_ref, aggb_ref, w1_ref, b1_ref, w2_ref,
                     b2_ref, gm_ref, bt_ref, out_ref):
    hn = _node_update(h_ref, agga_ref, aggb_ref, w1_ref, b1_ref, w2_ref,
                      b2_ref, gm_ref, bt_ref)

    @pl.when(pl.program_id(0) == 0)
    def _():
        out_ref[...] = jnp.zeros_like(out_ref)

    out_ref[...] += jnp.sum(hn, axis=0, keepdims=True)


_up_in_specs = [
    pl.BlockSpec((4, _BR, _HQ), lambda i: (0, i, 0)),
    pl.BlockSpec((2, _BR, _HQ), lambda i: (0, i, 0)),
    pl.BlockSpec((2, _BR, _HQ), lambda i: (0, i, 0)),
    pl.BlockSpec((_H, 2 * _H), lambda i: (0, 0)),
    pl.BlockSpec((1, 2 * _H), lambda i: (0, 0)),
    pl.BlockSpec((2 * _H, _H), lambda i: (0, 0)),
    pl.BlockSpec((1, _H), lambda i: (0, 0)),
    pl.BlockSpec((1, _H), lambda i: (0, 0)),
    pl.BlockSpec((1, _H), lambda i: (0, 0)),
]

_tc_up = pl.pallas_call(
    _tc_up_body,
    grid=(_N // _BR,),
    in_specs=_up_in_specs,
    out_specs=pl.BlockSpec((4, _BR, _HQ), lambda i: (0, i, 0)),
    out_shape=jax.ShapeDtypeStruct((4, _N, _HQ), jnp.float32),
)

_tc_up_last = pl.pallas_call(
    _tc_up_last_body,
    grid=(_N // _BR,),
    in_specs=_up_in_specs,
    out_specs=pl.BlockSpec((1, _H), lambda i: (0, 0)),
    out_shape=jax.ShapeDtypeStruct((1, _H), jnp.float32),
)


def _tc_head_body(g_ref, wr1_ref, br1_ref, wr2_ref, br2_ref, out_ref):
    t = jnp.maximum(jnp.dot(g_ref[...], wr1_ref[...],
                            preferred_element_type=jnp.float32)
                    + br1_ref[...], 0.0)
    out_ref[...] = jnp.dot(t, wr2_ref[...],
                           preferred_element_type=jnp.float32) + br2_ref[...]


_tc_head = pl.pallas_call(
    _tc_head_body,
    out_shape=jax.ShapeDtypeStruct((1, 1), jnp.float32),
)


def _edge_weights(We_l, be_l, q):
    zpad = jnp.zeros((3, _HQ), jnp.float32)
    lo = q * 2 * _HQ
    return jnp.concatenate(
        [We_l[:, lo:lo + _HQ], be_l[None, lo:lo + _HQ], zpad,
         We_l[:, lo + _HQ:lo + 2 * _HQ], be_l[None, lo + _HQ:lo + 2 * _HQ],
         zpad], axis=0)


def kernel(x, edge_index, edge_attr, W_in, b_in, We, be, W1, b1, W2, b2,
           gamma, beta, Wr1, br1, Wr2, br2):
    pad = _EPAD - _E
    src2d = jnp.concatenate(
        [edge_index[0], jnp.zeros((pad,), jnp.int32)]).reshape(_ROWS, 128)
    dst2d = jnp.concatenate(
        [edge_index[1], jnp.full((pad,), _N, jnp.int32)]).reshape(_ROWS, 128)
    ea_pad = jnp.pad(edge_attr.T, ((0, 0), (0, pad))).reshape(-1)

    h4 = _tc_in(x, W_in, b_in.reshape(1, _H))
    g = None
    for l in range(_L):
        hflat = h4.reshape(4 * _N, _HQ)
        agg_a = _sc_layer_q0(hflat, src2d, dst2d, ea_pad,
                             _edge_weights(We[l], be[l], 0))
        agg_b = _sc_layer_q1(hflat, src2d, dst2d, ea_pad,
                             _edge_weights(We[l], be[l], 1))
        args = (h4, agg_a.reshape(2, _AGGR, _HQ), agg_b.reshape(2, _AGGR, _HQ),
                W1[l], b1[l].reshape(1, -1), W2[l], b2[l].reshape(1, -1),
                gamma[l].reshape(1, -1), beta[l].reshape(1, -1))
        if l < _L - 1:
            h4 = _tc_up(*args)
        else:
            g = _tc_up_last(*args)
    out = _tc_head(g, Wr1, br1.reshape(1, -1), Wr2, br2.reshape(1, -1))
    return out.reshape(())
